# baseline (speedup 1.0000x reference)
import functools

import jax
import jax.numpy as jnp
import numpy as np
from jax import lax
from jax.experimental import pallas as pl
from jax.experimental.pallas import tpu as pltpu
from jax.experimental.pallas import tpu_sc as plsc

_VOCAB = 1000000
_D = 64
_B = 4096
_L = 200

_NC = 2
_NS = 16
_NW = _NC * _NS
_BB = _B // _NW
_NG = 4
_NO = 2


def _make_pe(d_model: int, max_len: int) -> np.ndarray:
    position = np.arange(max_len, dtype=np.float32)[:, None]
    div_term = np.exp(
        np.arange(0, d_model, 2, dtype=np.float32) * (-np.log(10000.0) / d_model)
    )
    pe = np.zeros((max_len, d_model), dtype=np.float32)
    pe[:, 0::2] = np.sin(position * div_term)
    pe[:, 1::2] = np.cos(position * div_term)
    return pe


_PE = _make_pe(_D, _L)


def _sc_embed(W, xT, pe):
    mesh = plsc.VectorSubcoreMesh(core_axis_name="c", subcore_axis_name="s")

    @functools.partial(
        pl.kernel,
        out_type=jax.ShapeDtypeStruct((_L, 8, _NW, 8, _BB), jnp.float32),
        mesh=mesh,
        scratch_types=[
            pltpu.VMEM((_L, _BB), jnp.int32),
            pltpu.VMEM((_NG, _BB, _D), jnp.float32),
            pltpu.VMEM((_NO, 8, 8, _BB), jnp.float32),
            pltpu.VMEM((_L * _D,), jnp.float32),
            pltpu.SemaphoreType.DMA,
            pltpu.SemaphoreType.DMA,
            pltpu.SemaphoreType.DMA,
            pltpu.SemaphoreType.DMA,
            pltpu.SemaphoreType.DMA,
            pltpu.SemaphoreType.DMA,
        ],
        compiler_params=pltpu.CompilerParams(
            use_tc_tiling_on_sc=False, needs_layout_passes=False
        ),
    )
    def body(w_hbm, xT_hbm, pe_hbm, out_hbm, idx_all, gbuf, obuf, pe_v,
             g0, g1, g2, g3, o0, o1):
        gs = (g0, g1, g2, g3)
        os_ = (o0, o1)
        wid = lax.axis_index("s") * _NC + lax.axis_index("c")
        b0 = wid * _BB
        pltpu.sync_copy(pe_hbm, pe_v)
        pltpu.sync_copy(xT_hbm.at[:, pl.ds(b0, _BB)], idx_all)

        iota = lax.iota(jnp.int32, 16)
        ridx = [i * 16 + iota for i in range(_BB // 16)]

        def fire_gather(l, g):
            pltpu.async_copy(w_hbm.at[idx_all.at[l]], gbuf.at[g], gs[g])

        def wait_gather(g):
            pltpu.make_async_copy(
                w_hbm.at[idx_all.at[0]], gbuf.at[g], gs[g]
            ).wait()

        def drain_store(o):
            pltpu.make_async_copy(
                obuf.at[o], out_hbm.at[0, :, 0], os_[o]
            ).wait()

        for g in range(_NG - 1):
            fire_gather(g, g)

        @pl.loop(0, _L, step=_NG)
        def _super(g_base):
            for j in range(_NG):
                l = g_base + j
                go = j
                oo = j % _NO
                wait_gather(go)

                @pl.when(l < _L - (_NG - 1))
                def _():
                    fire_gather(l + (_NG - 1), (j + _NG - 1) % _NG)

                @pl.when(l >= _NO)
                def _():
                    drain_store(oo)

                @pl.loop(0, 8)
                def _dblk(dblk):
                    for din in range(8):
                        d = dblk * 8 + din
                        pidx = jnp.full((16,), l * _D + d, jnp.int32)
                        pe_d = plsc.load_gather(pe_v, [pidx])
                        cidx = jnp.full((16,), d, jnp.int32)
                        for i in range(_BB // 16):
                            v = plsc.load_gather(gbuf.at[go], [ridx[i], cidx])
                            obuf[oo, dblk, din, pl.ds(i * 16, 16)] = v + pe_d

                pltpu.async_copy(
                    obuf.at[oo], out_hbm.at[l, :, wid], os_[oo]
                )

        for o in range(_NO):
            drain_store(o)

    return body(W, xT, pe)


def kernel(x, W):
    xT = x.T.astype(jnp.int32)
    pe = jnp.asarray(_PE.reshape(-1))
    out = _sc_embed(W, xT, pe)
    return (
        out.reshape(_L, 8, _NW, 8, _BB)
        .transpose(2, 4, 0, 1, 3)
        .reshape(_B, _L, _D)
    )


# speedup vs baseline: 1.7276x; 1.7276x over previous
import functools

import jax
import jax.numpy as jnp
import numpy as np
from jax import lax
from jax.experimental import pallas as pl
from jax.experimental.pallas import tpu as pltpu
from jax.experimental.pallas import tpu_sc as plsc

_VOCAB = 1000000
_D = 64
_B = 4096
_L = 200

_NC = 2
_NS = 16
_NW = _NC * _NS
_BB = _B // _NW
_LPB = 2
_NBLK = _L // _LPB
_NG = 3
_NO = 3


def _make_pe(d_model: int, max_len: int) -> np.ndarray:
    position = np.arange(max_len, dtype=np.float32)[:, None]
    div_term = np.exp(
        np.arange(0, d_model, 2, dtype=np.float32) * (-np.log(10000.0) / d_model)
    )
    pe = np.zeros((max_len, d_model), dtype=np.float32)
    pe[:, 0::2] = np.sin(position * div_term)
    pe[:, 1::2] = np.cos(position * div_term)
    return pe


_PE = _make_pe(_D, _L)


def _sc_embed(W, xT, pe):
    mesh = plsc.VectorSubcoreMesh(core_axis_name="c", subcore_axis_name="s")

    @functools.partial(
        pl.kernel,
        out_type=jax.ShapeDtypeStruct((_L, 8, _NW, 8, _BB), jnp.float32),
        mesh=mesh,
        scratch_types=[
            pltpu.VMEM((_NBLK, _LPB * _BB), jnp.int32),
            pltpu.VMEM((_NG, _LPB * _BB, _D), jnp.float32),
            pltpu.VMEM((_NO, 8, 8, _BB), jnp.float32),
            pltpu.VMEM((_L * _D,), jnp.float32),
        ] + [pltpu.SemaphoreType.DMA] * (_NG + _NO),
        compiler_params=pltpu.CompilerParams(
            use_tc_tiling_on_sc=False, needs_layout_passes=False
        ),
    )
    def body(w_hbm, xT_hbm, pe_hbm, out_hbm, idx_all, gbuf, obuf, pe_v,
             *sems):
        gs = sems[:_NG]
        os_ = sems[_NG:]
        wid = lax.axis_index("s") * _NC + lax.axis_index("c")
        b0 = wid * _BB
        pltpu.sync_copy(pe_hbm, pe_v)
        pltpu.sync_copy(xT_hbm.at[wid], idx_all)

        iota = lax.iota(jnp.int32, 16)
        ridx = [i * 16 + iota for i in range(_BB // 16)]

        def fire_gather(k, g):
            pltpu.async_copy(w_hbm.at[idx_all.at[k]], gbuf.at[g], gs[g])

        def wait_gather(g):
            pltpu.make_async_copy(
                w_hbm.at[idx_all.at[0]], gbuf.at[g], gs[g]
            ).wait()

        def drain_store(o):
            pltpu.make_async_copy(
                obuf.at[o], out_hbm.at[0, :, 0], os_[o]
            ).wait()

        def compute_store(l, go, sub, oo):
            @pl.when(l >= _NO)
            def _():
                drain_store(oo)

            @pl.loop(0, 8, unroll=2)
            def _dblk(dblk):
                for din in range(8):
                    d0 = dblk * 8 + din
                    cdiag = lax.bitwise_and(d0 + iota, _D - 1)
                    pe_d = plsc.load_gather(pe_v, [l * _D + cdiag])
                    dblk_v = lax.shift_right_logical(cdiag, 3)
                    din_v = lax.bitwise_and(cdiag, 7)
                    for i in range(_BB // 16):
                        v = plsc.load_gather(
                            gbuf.at[go],
                            [sub * _BB + ridx[i], cdiag],
                        )
                        plsc.store_scatter(
                            obuf.at[oo],
                            [dblk_v, din_v, ridx[i]],
                            v + pe_d,
                        )

            pltpu.async_copy(obuf.at[oo], out_hbm.at[l, :, wid], os_[oo])

        def do_block(k, j):
            go = j % _NG
            wait_gather(go)

            @pl.when(k + _NG - 1 < _NBLK)
            def _():
                fire_gather(k + _NG - 1, (j + _NG - 1) % _NG)

            for sub in range(_LPB):
                compute_store(k * _LPB + sub, go, sub, (2 * j + sub) % _NO)

        for g in range(_NG - 1):
            fire_gather(g, g)

        @pl.loop(0, _NBLK - 1, step=_NG)
        def _super(kb):
            for j in range(_NG):
                do_block(kb + j, j)

        do_block(_NBLK - 1, (_NBLK - 1) % _NG)
        for o in range(_NO):
            drain_store(o)

    return body(W, xT, pe)


def kernel(x, W):
    xT = (
        x.T.astype(jnp.int32)
        .reshape(_NBLK, _LPB, _NW, _BB)
        .transpose(2, 0, 1, 3)
        .reshape(_NW, _NBLK, _LPB * _BB)
    )
    pe = jnp.asarray(_PE.reshape(-1))
    out = _sc_embed(W, xT, pe)
    return (
        out.reshape(_L, 8, _NW, 8, _BB)
        .transpose(2, 4, 0, 1, 3)
        .reshape(_B, _L, _D)
    )


# speedup vs baseline: 2.6089x; 1.5101x over previous
import functools

import jax
import jax.numpy as jnp
import numpy as np
from jax import lax
from jax.experimental import pallas as pl
from jax.experimental.pallas import tpu as pltpu
from jax.experimental.pallas import tpu_sc as plsc

_VOCAB = 1000000
_D = 64
_B = 4096
_L = 200

_NC = 2
_NS = 16
_NW = _NC * _NS
_BB = _B // _NW
_LPB = 2
_NBLK = _L // _LPB
_NG = 3
_NO = 3


def _make_pe(d_model: int, max_len: int) -> np.ndarray:
    position = np.arange(max_len, dtype=np.float32)[:, None]
    div_term = np.exp(
        np.arange(0, d_model, 2, dtype=np.float32) * (-np.log(10000.0) / d_model)
    )
    pe = np.zeros((max_len, d_model), dtype=np.float32)
    pe[:, 0::2] = np.sin(position * div_term)
    pe[:, 1::2] = np.cos(position * div_term)
    return pe


_PE = _make_pe(_D, _L)


def _sc_embed(W, xT, pe):
    mesh = plsc.VectorSubcoreMesh(core_axis_name="c", subcore_axis_name="s")

    @functools.partial(
        pl.kernel,
        out_type=jax.ShapeDtypeStruct((_L, 8, _NW, 8, _BB), jnp.float32),
        mesh=mesh,
        scratch_types=[
            pltpu.VMEM((_NBLK, _LPB * _BB), jnp.int32),
            pltpu.VMEM((_NG, _LPB * _BB, _D), jnp.float32),
            pltpu.VMEM((_NO, 8, 8, _BB), jnp.float32),
            pltpu.VMEM((_L * _D,), jnp.float32),
        ] + [pltpu.SemaphoreType.DMA] * (_NG + _NO),
        compiler_params=pltpu.CompilerParams(
            use_tc_tiling_on_sc=False, needs_layout_passes=False
        ),
    )
    def body(w_hbm, xT_hbm, pe_hbm, out_hbm, idx_all, gbuf, obuf, pe_v,
             *sems):
        gs = sems[:_NG]
        os_ = sems[_NG:]
        wid = lax.axis_index("s") * _NC + lax.axis_index("c")
        b0 = wid * _BB
        pltpu.sync_copy(pe_hbm, pe_v)
        pltpu.sync_copy(xT_hbm.at[wid], idx_all)

        iota = lax.iota(jnp.int32, 16)
        ridx = [i * 16 + iota for i in range(_BB // 16)]

        def fire_gather(k, g):
            pltpu.async_copy(w_hbm.at[idx_all.at[k]], gbuf.at[g], gs[g])

        def wait_gather(g):
            pltpu.make_async_copy(
                w_hbm.at[idx_all.at[0]], gbuf.at[g], gs[g]
            ).wait()

        def drain_store(o):
            pltpu.make_async_copy(
                obuf.at[o], out_hbm.at[0, :, 0], os_[o]
            ).wait()

        def compute_store(l, go, sub, oo):
            @pl.when(l >= _NO)
            def _():
                drain_store(oo)

            @functools.partial(plsc.parallel_loop, 0, 8, unroll=2)
            def _dblk(dblk):
                for din in range(8):
                    d0 = dblk * 8 + din
                    cdiag = lax.bitwise_and(d0 + iota, _D - 1)
                    pe_d = plsc.load_gather(pe_v, [l * _D + cdiag])
                    dblk_v = lax.shift_right_logical(cdiag, 3)
                    din_v = lax.bitwise_and(cdiag, 7)
                    for i in range(_BB // 16):
                        v = plsc.load_gather(
                            gbuf.at[go],
                            [sub * _BB + ridx[i], cdiag],
                        )
                        plsc.store_scatter(
                            obuf.at[oo],
                            [dblk_v, din_v, ridx[i]],
                            v + pe_d,
                        )

            pltpu.async_copy(obuf.at[oo], out_hbm.at[l, :, wid], os_[oo])

        def do_block(k, j):
            go = j % _NG
            wait_gather(go)

            @pl.when(k + _NG - 1 < _NBLK)
            def _():
                fire_gather(k + _NG - 1, (j + _NG - 1) % _NG)

            for sub in range(_LPB):
                compute_store(k * _LPB + sub, go, sub, (2 * j + sub) % _NO)

        for g in range(_NG - 1):
            fire_gather(g, g)

        @pl.loop(0, _NBLK - 1, step=_NG)
        def _super(kb):
            for j in range(_NG):
                do_block(kb + j, j)

        do_block(_NBLK - 1, (_NBLK - 1) % _NG)
        for o in range(_NO):
            drain_store(o)

    return body(W, xT, pe)


def kernel(x, W):
    xT = (
        x.T.astype(jnp.int32)
        .reshape(_NBLK, _LPB, _NW, _BB)
        .transpose(2, 0, 1, 3)
        .reshape(_NW, _NBLK, _LPB * _BB)
    )
    pe = jnp.asarray(_PE.reshape(-1))
    out = _sc_embed(W, xT, pe)
    return (
        out.reshape(_L, 8, _NW, 8, _BB)
        .transpose(2, 4, 0, 1, 3)
        .reshape(_B, _L, _D)
    )
